# baseline (speedup 1.0000x reference)
import jax
import jax.numpy as jnp
from jax.experimental import pallas as pl

_DEPTH = 1000
_BR = 16


def _one_hot_body(idx_ref, out_ref):
    idx = idx_ref[...]
    d = jax.lax.broadcasted_iota(jnp.int32, (idx.shape[0], idx.shape[1], _DEPTH), 2)
    out_ref[...] = (idx[:, :, None] == d).astype(jnp.float32)


def kernel(inputs):
    n, c = inputs.shape
    idx = inputs.astype(jnp.int32)
    return pl.pallas_call(
        _one_hot_body,
        grid=(n // _BR,),
        in_specs=[pl.BlockSpec((_BR, c), lambda i: (i, 0))],
        out_specs=pl.BlockSpec((_BR, c, _DEPTH), lambda i: (i, 0, 0)),
        out_shape=jax.ShapeDtypeStruct((n, c, _DEPTH), jnp.float32),
    )(idx)


# speedup vs baseline: 1.0974x; 1.0974x over previous
import jax
import jax.numpy as jnp
from jax.experimental import pallas as pl

_DEPTH = 1000
_BR = 64


def _one_hot_body(idx_ref, out_ref):
    idx = idx_ref[...]
    d = jax.lax.broadcasted_iota(jnp.int32, (idx.shape[0], idx.shape[1], _DEPTH), 2)
    out_ref[...] = (idx[:, :, None] == d).astype(jnp.float32)


def kernel(inputs):
    n, c = inputs.shape
    idx = inputs.astype(jnp.int32)
    return pl.pallas_call(
        _one_hot_body,
        grid=(n // _BR,),
        in_specs=[pl.BlockSpec((_BR, c), lambda i: (i, 0))],
        out_specs=pl.BlockSpec((_BR, c, _DEPTH), lambda i: (i, 0, 0)),
        out_shape=jax.ShapeDtypeStruct((n, c, _DEPTH), jnp.float32),
    )(idx)


# speedup vs baseline: 1.0981x; 1.0007x over previous
import jax
import jax.numpy as jnp
from jax.experimental import pallas as pl
from jax.experimental.pallas import tpu as pltpu

_DEPTH = 1000
_BR = 32
_NBUF = 4


def _one_hot_body(idx_ref, out_hbm, buf, sem):
    i = pl.program_id(0)
    ng = pl.num_programs(0)
    slot = jax.lax.rem(i, _NBUF)

    @pl.when(i >= _NBUF)
    def _wait_prev():
        prev = i - _NBUF
        pltpu.make_async_copy(
            buf.at[slot], out_hbm.at[pl.ds(prev * _BR, _BR)], sem.at[slot]
        ).wait()

    idx = idx_ref[...]
    d = jax.lax.broadcasted_iota(jnp.int32, (idx.shape[0], idx.shape[1], _DEPTH), 2)
    buf[slot] = (idx[:, :, None] == d).astype(jnp.float32)

    pltpu.make_async_copy(
        buf.at[slot], out_hbm.at[pl.ds(i * _BR, _BR)], sem.at[slot]
    ).start()

    @pl.when(i == ng - 1)
    def _drain():
        for k in range(_NBUF):
            step = ng - _NBUF + k
            s = jax.lax.rem(jnp.int32(step), _NBUF)
            pltpu.make_async_copy(
                buf.at[s], out_hbm.at[pl.ds(step * _BR, _BR)], sem.at[s]
            ).wait()


def kernel(inputs):
    n, c = inputs.shape
    idx = inputs.astype(jnp.int32)
    return pl.pallas_call(
        _one_hot_body,
        grid=(n // _BR,),
        in_specs=[pl.BlockSpec((_BR, c), lambda i: (i, 0))],
        out_specs=pl.BlockSpec(memory_space=pl.ANY),
        out_shape=jax.ShapeDtypeStruct((n, c, _DEPTH), jnp.float32),
        scratch_shapes=[
            pltpu.VMEM((_NBUF, _BR, c, _DEPTH), jnp.float32),
            pltpu.SemaphoreType.DMA((_NBUF,)),
        ],
    )(idx)
